# baseline (speedup 1.0000x reference)
import functools

import jax
import jax.numpy as jnp
from jax import lax
from jax.experimental import pallas as pl
from jax.experimental.pallas import tpu as pltpu
from jax.experimental.pallas import tpu_sc as plsc

BATCH = 4096
HIDDEN = 64
CHUNK = 512


def _make_sc_gather(V, D, B):
    info = plsc.get_sparse_core_info()
    NC = info.num_cores
    per_core = B // NC
    n_chunks = per_core // CHUNK
    mesh = plsc.ScalarSubcoreMesh(axis_name="c", num_cores=NC)

    @functools.partial(
        pl.kernel,
        mesh=mesh,
        out_type=jax.ShapeDtypeStruct((B, D), jnp.float32),
        scratch_types=[
            pltpu.SMEM((CHUNK,), jnp.int32),
            pltpu.SemaphoreType.DMA,
        ],
    )
    def gather(table_hbm, idx_hbm, out_hbm, idx_s, sem):
        core = lax.axis_index("c")
        base = core * per_core
        for c in range(n_chunks):
            cbase = base + c * CHUNK
            pltpu.sync_copy(idx_hbm.at[pl.ds(cbase, CHUNK)], idx_s)

            def fire(j, carry, cbase=cbase):
                i = idx_s[j]
                pltpu.make_async_copy(
                    table_hbm.at[pl.ds(i, 1)],
                    out_hbm.at[pl.ds(cbase + j, 1)],
                    sem,
                ).start()
                return carry

            lax.fori_loop(0, CHUNK, fire, 0)
            pltpu.make_async_copy(
                table_hbm.at[pl.ds(0, CHUNK)],
                out_hbm.at[pl.ds(cbase, CHUNK)],
                sem,
            ).wait()

    return gather


def _gru_body(x_ref, h_ref, wih_ref, whh_ref, bih_ref, bhh_ref, out_ref,
              hid_ref):
    H = HIDDEN
    x = x_ref[...]
    h = h_ref[0]
    dims = (((1,), (1,)), ((), ()))
    gi = lax.dot_general(x, wih_ref[...], dims,
                         preferred_element_type=jnp.float32)
    gh = lax.dot_general(h, whh_ref[...], dims,
                         preferred_element_type=jnp.float32)
    gi = gi + bih_ref[...].reshape(1, 3 * H)
    gh = gh + bhh_ref[...].reshape(1, 3 * H)
    r = jax.nn.sigmoid(gi[:, :H] + gh[:, :H])
    z = jax.nn.sigmoid(gi[:, H:2 * H] + gh[:, H:2 * H])
    n = jnp.tanh(gi[:, 2 * H:] + r * gh[:, 2 * H:])
    h1 = (1.0 - z) * n + z * h
    out_ref[0] = h1
    hid_ref[0] = h1


def kernel(input_data, batch_size, hidden, embedding_matrix, W_ih, W_hh,
           b_ih, b_hh):
    V, D = embedding_matrix.shape
    idx = input_data.astype(jnp.int32)

    gather = _make_sc_gather(V, D, BATCH)
    x = gather(embedding_matrix, idx)

    out, hid = pl.pallas_call(
        _gru_body,
        out_shape=(
            jax.ShapeDtypeStruct((1, BATCH, HIDDEN), jnp.float32),
            jax.ShapeDtypeStruct((1, BATCH, HIDDEN), jnp.float32),
        ),
    )(x, hidden, W_ih, W_hh, b_ih, b_hh)
    return (out, hid)


# speedup vs baseline: 1.0090x; 1.0090x over previous
import jax
import jax.numpy as jnp
from jax import lax
from jax.experimental import pallas as pl
from jax.experimental.pallas import tpu as pltpu

BATCH = 4096
HIDDEN = 64
K = 8
LANES = 128


def _body(idx_ref, *refs):
    (t0, t1, t2, t3, t4, t5, t6, t7, h_ref, wih_ref, whh_ref, bih_ref,
     bhh_ref, out_ref, hid_ref) = refs
    H = HIDDEN
    j = pl.program_id(0)
    rs = jnp.array([idx_ref[K * j + k] % LANES for k in range(K)],
                   dtype=jnp.int32)
    tiles = jnp.concatenate(
        [t0[...], t1[...], t2[...], t3[...], t4[...], t5[...], t6[...],
         t7[...]], axis=1)
    m = jax.lax.broadcasted_iota(jnp.int32, (K * LANES, K), 0)
    col = jax.lax.broadcasted_iota(jnp.int32, (K * LANES, K), 1)
    sel = (m == col * LANES + rs[None, :]).astype(jnp.float32)
    x = lax.dot_general(sel, tiles, (((0,), (1,)), ((), ())),
                        preferred_element_type=jnp.float32)
    h = h_ref[0]
    gi = lax.dot_general(x, wih_ref[...], (((1,), (1,)), ((), ())),
                         preferred_element_type=jnp.float32)
    gh = lax.dot_general(h, whh_ref[...], (((1,), (1,)), ((), ())),
                         preferred_element_type=jnp.float32)
    gi = gi + bih_ref[...].reshape(1, 3 * H)
    gh = gh + bhh_ref[...].reshape(1, 3 * H)
    r = jax.nn.sigmoid(gi[:, :H] + gh[:, :H])
    z = jax.nn.sigmoid(gi[:, H:2 * H] + gh[:, H:2 * H])
    n = jnp.tanh(gi[:, 2 * H:] + r * gh[:, 2 * H:])
    h1 = (1.0 - z) * n + z * h
    out_ref[0] = h1
    hid_ref[0] = h1


def kernel(input_data, batch_size, hidden, embedding_matrix, W_ih, W_hh,
           b_ih, b_hh):
    V, D = embedding_matrix.shape
    idx = input_data.astype(jnp.int32)
    tablet = embedding_matrix.T

    n_steps = BATCH // K
    table_specs = [
        pl.BlockSpec((D, LANES),
                     (lambda j, idx_ref, k=k: (0, idx_ref[K * j + k] // LANES)))
        for k in range(K)
    ]
    grid_spec = pltpu.PrefetchScalarGridSpec(
        num_scalar_prefetch=1,
        grid=(n_steps,),
        in_specs=table_specs + [
            pl.BlockSpec((1, K, HIDDEN), lambda j, idx_ref: (0, j, 0)),
            pl.BlockSpec((3 * HIDDEN, D), lambda j, idx_ref: (0, 0)),
            pl.BlockSpec((3 * HIDDEN, HIDDEN), lambda j, idx_ref: (0, 0)),
            pl.BlockSpec((3 * HIDDEN,), lambda j, idx_ref: (0,)),
            pl.BlockSpec((3 * HIDDEN,), lambda j, idx_ref: (0,)),
        ],
        out_specs=[
            pl.BlockSpec((1, K, HIDDEN), lambda j, idx_ref: (0, j, 0)),
            pl.BlockSpec((1, K, HIDDEN), lambda j, idx_ref: (0, j, 0)),
        ],
    )
    out, hid = pl.pallas_call(
        _body,
        grid_spec=grid_spec,
        out_shape=(
            jax.ShapeDtypeStruct((1, BATCH, HIDDEN), jnp.float32),
            jax.ShapeDtypeStruct((1, BATCH, HIDDEN), jnp.float32),
        ),
    )(idx, tablet, tablet, tablet, tablet, tablet, tablet, tablet, tablet,
      hidden, W_ih, W_hh, b_ih, b_hh)
    return (out, hid)


# speedup vs baseline: 3.7150x; 3.6817x over previous
import functools

import jax
import jax.numpy as jnp
from jax import lax
from jax.experimental import pallas as pl
from jax.experimental.pallas import tpu as pltpu
from jax.experimental.pallas import tpu_sc as plsc

BATCH = 4096
HIDDEN = 64
LANES = 128


def _make_sc_gather(V, D, B):
    info = plsc.get_sparse_core_info()
    NC, NS = info.num_cores, info.num_subcores
    NW = NC * NS
    assert B % (8 * NW) == 0
    b_per_w = B // NW
    L = 16
    mesh = plsc.VectorSubcoreMesh(core_axis_name="c", subcore_axis_name="s")

    @functools.partial(
        pl.kernel,
        mesh=mesh,
        out_type=jax.ShapeDtypeStruct((B, D), jnp.float32),
        scratch_types=[
            pltpu.VMEM((b_per_w,), jnp.int32),
            pltpu.VMEM((D, LANES), jnp.float32),
            pltpu.VMEM((D, LANES), jnp.float32),
            pltpu.VMEM((b_per_w, D), jnp.float32),
            pltpu.SemaphoreType.DMA,
            pltpu.SemaphoreType.DMA,
        ],
        compiler_params=pltpu.CompilerParams(needs_layout_passes=False),
    )
    def gather(tablet_hbm, idx_hbm, out_hbm, idx_v, buf0, buf1, rows_v,
               sem0, sem1):
        wid = lax.axis_index("s") * NC + lax.axis_index("c")
        base = wid * b_per_w
        pltpu.sync_copy(idx_hbm.at[pl.ds(base, b_per_w)], idx_v)
        lane = lax.iota(jnp.int32, L)
        bufs = (buf0, buf1)
        sems = (sem0, sem1)

        def fetch(j):
            vec = idx_v[pl.ds((j // L) * L, L)]
            i = jnp.sum(jnp.where(lane == (j % L), vec, 0))
            off = pl.multiple_of((i >> 7) * LANES, LANES)
            d = pltpu.make_async_copy(
                tablet_hbm.at[:, pl.ds(off, LANES)], bufs[j % 2],
                sems[j % 2])
            d.start()
            return i, d

        nxt = fetch(0)
        for j in range(b_per_w):
            i, d = nxt
            if j + 1 < b_per_w:
                nxt = fetch(j + 1)
            d.wait()
            r = jnp.full((L,), i & (LANES - 1), dtype=jnp.int32)
            buf = bufs[j % 2]
            for q in range(D // L):
                vals = plsc.load_gather(
                    buf, [lax.iota(jnp.int32, L) + q * L, r])
                rows_v[j, pl.ds(q * L, L)] = vals
        pltpu.sync_copy(rows_v, out_hbm.at[pl.ds(base, b_per_w)])

    return gather


def _gru_body(x_ref, h_ref, wih_ref, whh_ref, bih_ref, bhh_ref, out_ref,
              hid_ref):
    H = HIDDEN
    x = x_ref[...]
    h = h_ref[0]
    dims = (((1,), (1,)), ((), ()))
    gi = lax.dot_general(x, wih_ref[...], dims,
                         preferred_element_type=jnp.float32)
    gh = lax.dot_general(h, whh_ref[...], dims,
                         preferred_element_type=jnp.float32)
    gi = gi + bih_ref[...].reshape(1, 3 * H)
    gh = gh + bhh_ref[...].reshape(1, 3 * H)
    r = jax.nn.sigmoid(gi[:, :H] + gh[:, :H])
    z = jax.nn.sigmoid(gi[:, H:2 * H] + gh[:, H:2 * H])
    n = jnp.tanh(gi[:, 2 * H:] + r * gh[:, 2 * H:])
    h1 = (1.0 - z) * n + z * h
    out_ref[0] = h1
    hid_ref[0] = h1


def kernel(input_data, batch_size, hidden, embedding_matrix, W_ih, W_hh,
           b_ih, b_hh):
    V, D = embedding_matrix.shape
    idx = input_data.astype(jnp.int32)
    tablet = embedding_matrix.T

    gather = _make_sc_gather(V, D, BATCH)
    x = gather(tablet, idx)

    out, hid = pl.pallas_call(
        _gru_body,
        out_shape=(
            jax.ShapeDtypeStruct((1, BATCH, HIDDEN), jnp.float32),
            jax.ShapeDtypeStruct((1, BATCH, HIDDEN), jnp.float32),
        ),
    )(x, hidden, W_ih, W_hh, b_ih, b_hh)
    return (out, hid)


# speedup vs baseline: 4.5411x; 1.2224x over previous
import functools

import jax
import jax.numpy as jnp
from jax import lax
from jax.experimental import pallas as pl
from jax.experimental.pallas import tpu as pltpu
from jax.experimental.pallas import tpu_sc as plsc

BATCH = 4096
HIDDEN = 64
LANES = 128


def _make_sc_gather(V, D, B):
    info = plsc.get_sparse_core_info()
    NC, NS = info.num_cores, info.num_subcores
    NW = NC * NS
    assert B % (8 * NW) == 0
    b_per_w = B // NW
    L = 16
    mesh = plsc.VectorSubcoreMesh(core_axis_name="c", subcore_axis_name="s")

    @functools.partial(
        pl.kernel,
        mesh=mesh,
        out_type=jax.ShapeDtypeStruct((B, D), jnp.float32),
        scratch_types=[
            pltpu.VMEM((b_per_w,), jnp.int32),
            pltpu.VMEM((D, LANES), jnp.float32),
            pltpu.VMEM((D, LANES), jnp.float32),
            pltpu.VMEM((D, LANES), jnp.float32),
            pltpu.VMEM((D, LANES), jnp.float32),
            pltpu.VMEM((b_per_w, D), jnp.float32),
            pltpu.SemaphoreType.DMA,
            pltpu.SemaphoreType.DMA,
            pltpu.SemaphoreType.DMA,
            pltpu.SemaphoreType.DMA,
        ],
        compiler_params=pltpu.CompilerParams(needs_layout_passes=False),
    )
    def gather(tablet_hbm, idx_hbm, out_hbm, idx_v, buf0, buf1, buf2, buf3,
               rows_v, sem0, sem1, sem2, sem3):
        NBUF = 4
        wid = lax.axis_index("s") * NC + lax.axis_index("c")
        base = wid * b_per_w
        pltpu.sync_copy(idx_hbm.at[pl.ds(base, b_per_w)], idx_v)
        lane = lax.iota(jnp.int32, L)
        bufs = (buf0, buf1, buf2, buf3)
        sems = (sem0, sem1, sem2, sem3)

        def fetch(j):
            vec = idx_v[pl.ds((j // L) * L, L)]
            i = jnp.sum(jnp.where(lane == (j % L), vec, 0))
            off = pl.multiple_of((i >> 7) * LANES, LANES)
            d = pltpu.make_async_copy(
                tablet_hbm.at[:, pl.ds(off, LANES)], bufs[j % NBUF],
                sems[j % NBUF])
            d.start()
            return i, d

        pending = [fetch(j) for j in range(NBUF - 1)]
        for j in range(b_per_w):
            i, d = pending.pop(0)
            if j + NBUF - 1 < b_per_w:
                pending.append(fetch(j + NBUF - 1))
            d.wait()
            r = jnp.full((L,), i & (LANES - 1), dtype=jnp.int32)
            buf = bufs[j % NBUF]
            for q in range(D // L):
                vals = plsc.load_gather(
                    buf, [lax.iota(jnp.int32, L) + q * L, r])
                rows_v[j, pl.ds(q * L, L)] = vals
        pltpu.sync_copy(rows_v, out_hbm.at[pl.ds(base, b_per_w)])

    return gather


def _gru_body(x_ref, h_ref, wih_ref, whh_ref, bih_ref, bhh_ref, out_ref,
              hid_ref):
    H = HIDDEN
    x = x_ref[...]
    h = h_ref[0]
    dims = (((1,), (1,)), ((), ()))
    gi = lax.dot_general(x, wih_ref[...], dims,
                         preferred_element_type=jnp.float32)
    gh = lax.dot_general(h, whh_ref[...], dims,
                         preferred_element_type=jnp.float32)
    gi = gi + bih_ref[...].reshape(1, 3 * H)
    gh = gh + bhh_ref[...].reshape(1, 3 * H)
    r = jax.nn.sigmoid(gi[:, :H] + gh[:, :H])
    z = jax.nn.sigmoid(gi[:, H:2 * H] + gh[:, H:2 * H])
    n = jnp.tanh(gi[:, 2 * H:] + r * gh[:, 2 * H:])
    h1 = (1.0 - z) * n + z * h
    out_ref[0] = h1
    hid_ref[0] = h1


def kernel(input_data, batch_size, hidden, embedding_matrix, W_ih, W_hh,
           b_ih, b_hh):
    V, D = embedding_matrix.shape
    idx = input_data.astype(jnp.int32)
    tablet = embedding_matrix.T

    gather = _make_sc_gather(V, D, BATCH)
    x = gather(tablet, idx)

    out, hid = pl.pallas_call(
        _gru_body,
        out_shape=(
            jax.ShapeDtypeStruct((1, BATCH, HIDDEN), jnp.float32),
            jax.ShapeDtypeStruct((1, BATCH, HIDDEN), jnp.float32),
        ),
    )(x, hidden, W_ih, W_hh, b_ih, b_hh)
    return (out, hid)


# speedup vs baseline: 5.0531x; 1.1127x over previous
import functools

import jax
import jax.numpy as jnp
from jax import lax
from jax.experimental import pallas as pl
from jax.experimental.pallas import tpu as pltpu
from jax.experimental.pallas import tpu_sc as plsc

BATCH = 4096
HIDDEN = 64
LANES = 128


def _make_sc_gather(V, D, B):
    info = plsc.get_sparse_core_info()
    NC, NS = info.num_cores, info.num_subcores
    NW = NC * NS
    assert B % (8 * NW) == 0
    b_per_w = B // NW
    L = 16
    mesh = plsc.VectorSubcoreMesh(core_axis_name="c", subcore_axis_name="s")

    @functools.partial(
        pl.kernel,
        mesh=mesh,
        out_type=jax.ShapeDtypeStruct((B, D), jnp.float32),
        scratch_types=[
            pltpu.VMEM((b_per_w,), jnp.int32),
            pltpu.VMEM((D, LANES), jnp.float32),
            pltpu.VMEM((D, LANES), jnp.float32),
            pltpu.VMEM((D, LANES), jnp.float32),
            pltpu.VMEM((D, LANES), jnp.float32),
            pltpu.VMEM((D, LANES), jnp.float32),
            pltpu.VMEM((D, LANES), jnp.float32),
            pltpu.VMEM((D, LANES), jnp.float32),
            pltpu.VMEM((D, LANES), jnp.float32),
            pltpu.VMEM((b_per_w, D), jnp.float32),
            pltpu.SemaphoreType.DMA,
            pltpu.SemaphoreType.DMA,
            pltpu.SemaphoreType.DMA,
            pltpu.SemaphoreType.DMA,
            pltpu.SemaphoreType.DMA,
            pltpu.SemaphoreType.DMA,
            pltpu.SemaphoreType.DMA,
            pltpu.SemaphoreType.DMA,
        ],
        compiler_params=pltpu.CompilerParams(needs_layout_passes=False),
    )
    def gather(tablet_hbm, idx_hbm, out_hbm, idx_v, buf0, buf1, buf2, buf3,
               buf4, buf5, buf6, buf7, rows_v, sem0, sem1, sem2, sem3, sem4,
               sem5, sem6, sem7):
        NBUF = 8
        wid = lax.axis_index("s") * NC + lax.axis_index("c")
        base = wid * b_per_w
        pltpu.sync_copy(idx_hbm.at[pl.ds(base, b_per_w)], idx_v)
        lane = lax.iota(jnp.int32, L)
        bufs = (buf0, buf1, buf2, buf3, buf4, buf5, buf6, buf7)
        sems = (sem0, sem1, sem2, sem3, sem4, sem5, sem6, sem7)

        def fetch(j):
            vec = idx_v[pl.ds((j // L) * L, L)]
            i = jnp.sum(jnp.where(lane == (j % L), vec, 0))
            off = pl.multiple_of((i >> 7) * LANES, LANES)
            d = pltpu.make_async_copy(
                tablet_hbm.at[:, pl.ds(off, LANES)], bufs[j % NBUF],
                sems[j % NBUF])
            d.start()
            return i, d

        pending = [fetch(j) for j in range(NBUF - 1)]
        for j in range(b_per_w):
            i, d = pending.pop(0)
            if j + NBUF - 1 < b_per_w:
                pending.append(fetch(j + NBUF - 1))
            d.wait()
            r = jnp.full((L,), i & (LANES - 1), dtype=jnp.int32)
            buf = bufs[j % NBUF]
            for q in range(D // L):
                vals = plsc.load_gather(
                    buf, [lax.iota(jnp.int32, L) + q * L, r])
                rows_v[j, pl.ds(q * L, L)] = vals
        pltpu.sync_copy(rows_v, out_hbm.at[pl.ds(base, b_per_w)])

    return gather


def _gru_body(x_ref, h_ref, wih_ref, whh_ref, bih_ref, bhh_ref, out_ref,
              hid_ref):
    H = HIDDEN
    x = x_ref[...]
    h = h_ref[0]
    dims = (((1,), (1,)), ((), ()))
    gi = lax.dot_general(x, wih_ref[...], dims,
                         preferred_element_type=jnp.float32)
    gh = lax.dot_general(h, whh_ref[...], dims,
                         preferred_element_type=jnp.float32)
    gi = gi + bih_ref[...].reshape(1, 3 * H)
    gh = gh + bhh_ref[...].reshape(1, 3 * H)
    r = jax.nn.sigmoid(gi[:, :H] + gh[:, :H])
    z = jax.nn.sigmoid(gi[:, H:2 * H] + gh[:, H:2 * H])
    n = jnp.tanh(gi[:, 2 * H:] + r * gh[:, 2 * H:])
    h1 = (1.0 - z) * n + z * h
    out_ref[0] = h1
    hid_ref[0] = h1


def kernel(input_data, batch_size, hidden, embedding_matrix, W_ih, W_hh,
           b_ih, b_hh):
    V, D = embedding_matrix.shape
    idx = input_data.astype(jnp.int32)
    tablet = embedding_matrix.T

    gather = _make_sc_gather(V, D, BATCH)
    x = gather(tablet, idx)

    out, hid = pl.pallas_call(
        _gru_body,
        out_shape=(
            jax.ShapeDtypeStruct((1, BATCH, HIDDEN), jnp.float32),
            jax.ShapeDtypeStruct((1, BATCH, HIDDEN), jnp.float32),
        ),
    )(x, hidden, W_ih, W_hh, b_ih, b_hh)
    return (out, hid)
